# baseline (speedup 1.0000x reference)
import functools

import jax
import jax.numpy as jnp
from jax import lax
from jax.experimental import pallas as pl
from jax.experimental.pallas import tpu as pltpu
from jax.experimental.pallas import tpu_sc as plsc

N = 50000
D = 64
NUM_LAYERS = 3
E = 800000

NC = 2
NS = 16
LANES = 16

HALF = N // NC
ACC_ROWS = 25088
DUMP = HALF
SLAB = ACC_ROWS // NS
TAIL = HALF - (NS - 1) * SLAB

CH = 128
K = CH // 128
NCHUNK = 394
NPAIRS = NCHUNK // 2
EPT = NCHUNK * CH
E_PAD = EPT * NS

_mesh = plsc.VectorSubcoreMesh(
    core_axis_name="c", subcore_axis_name="s", num_cores=NC, num_subcores=NS)
_sc_params = pltpu.CompilerParams(use_tc_tiling_on_sc=False)


def _compute_lidx(didx, lidx, base):
    for k in range(K):
        for v in range(128 // LANES):
            d = didx[pl.ds(k * 128 + v * LANES, LANES)]
            loc = d - base
            ok = (loc >= 0) & (loc < HALF)
            lidx[k, pl.ds(v * LANES, LANES)] = jnp.where(
                ok, loc, DUMP + (d & 63))


def _deg_body(dst_hbm, deg_hbm, didx, lidx, ones_v, zrow, dacc, sem):
    cid = lax.axis_index("c")
    sid = lax.axis_index("s")
    base = cid * HALF

    ov = jnp.ones((LANES,), jnp.float32)
    for v in range(128 // LANES):
        ones_v[pl.ds(v * LANES, LANES)] = ov
    zv = jnp.zeros((LANES,), jnp.float32)

    def zbody(i, _):
        zrow[pl.ds(i * LANES, LANES)] = zv
        return 0

    lax.fori_loop(0, SLAB // LANES, zbody, 0, unroll=4)
    pltpu.sync_copy(zrow, dacc.at[pl.ds(sid * SLAB, SLAB)])
    plsc.subcore_barrier()

    ebase = sid * EPT

    def chunk(j, _):
        off = ebase + j * CH
        pltpu.sync_copy(dst_hbm.at[pl.ds(off, CH)], didx)
        _compute_lidx(didx, lidx, base)
        for k in range(K):
            pltpu.sync_copy(ones_v, dacc.at[lidx.at[k]], add=True)
        return 0

    lax.fori_loop(0, NCHUNK, chunk, 0)
    plsc.subcore_barrier()

    @pl.when(sid < NS - 1)
    def _():
        pltpu.sync_copy(dacc.at[pl.ds(sid * SLAB, SLAB)], zrow)
        pltpu.sync_copy(zrow, deg_hbm.at[pl.ds(base + sid * SLAB, SLAB)])

    @pl.when(sid == NS - 1)
    def _():
        pltpu.sync_copy(dacc.at[pl.ds((NS - 1) * SLAB, TAIL)],
                        zrow.at[pl.ds(0, TAIL)])
        pltpu.sync_copy(zrow.at[pl.ds(0, TAIL)],
                        deg_hbm.at[pl.ds(base + (NS - 1) * SLAB, TAIL)])


_deg_call = functools.partial(
    pl.kernel,
    out_type=jax.ShapeDtypeStruct((N,), jnp.float32),
    mesh=_mesh,
    compiler_params=_sc_params,
    scratch_types=[
        pltpu.VMEM((CH,), jnp.int32),
        pltpu.VMEM((K, 128), jnp.int32),
        pltpu.VMEM((128,), jnp.float32),
        pltpu.VMEM((SLAB,), jnp.float32),
        pltpu.VMEM_SHARED((ACC_ROWS,), jnp.float32),
        pltpu.SemaphoreType.DMA,
    ],
)(_deg_body)


def _prop_body(z_hbm, src_hbm, dst_hbm, out_hbm,
               sidx_a, sidx_b, didx_a, didx_b, lidx_a, lidx_b,
               rows_a, rows_b, acc, sem_a, sem_b):
    cid = lax.axis_index("c")
    sid = lax.axis_index("s")
    base = cid * HALF

    zv = jnp.zeros((LANES,), jnp.float32)

    def zbody(i, _):
        for c in range(D // LANES):
            rows_a[i, pl.ds(c * LANES, LANES)] = zv
        return 0

    lax.fori_loop(0, CH, zbody, 0, unroll=4)
    for k in range(SLAB // CH):
        pltpu.sync_copy(rows_a, acc.at[pl.ds(sid * SLAB + k * CH, CH)])
    rem = SLAB - (SLAB // CH) * CH
    if rem:
        pltpu.sync_copy(rows_a.at[pl.ds(0, rem)],
                        acc.at[pl.ds(sid * SLAB + (SLAB // CH) * CH, rem)])
    plsc.subcore_barrier()

    ebase = sid * EPT

    def load_idx(j, sidx, didx, lidx):
        off = ebase + j * CH
        pltpu.sync_copy(src_hbm.at[pl.ds(off, CH)], sidx)
        pltpu.sync_copy(dst_hbm.at[pl.ds(off, CH)], didx)
        for v in range(CH // LANES):
            d = didx[pl.ds(v * LANES, LANES)]
            loc = d - base
            ok = (loc >= 0) & (loc < HALF)
            lidx[0, pl.ds(v * LANES, LANES)] = jnp.where(
                ok, loc, DUMP + (d & 63))

    load_idx(0, sidx_a, didx_a, lidx_a)
    pltpu.async_copy(z_hbm.at[sidx_a], rows_a, sem_a)

    def pair(t, _):
        j0 = 2 * t
        load_idx(j0 + 1, sidx_b, didx_b, lidx_b)
        pltpu.make_async_copy(z_hbm.at[sidx_a], rows_a, sem_a).wait()
        pltpu.async_copy(z_hbm.at[sidx_b], rows_b, sem_b)
        pltpu.sync_copy(rows_a, acc.at[lidx_a.at[0]], add=True)

        @pl.when(t < NPAIRS - 1)
        def _():
            load_idx(j0 + 2, sidx_a, didx_a, lidx_a)

        pltpu.make_async_copy(z_hbm.at[sidx_b], rows_b, sem_b).wait()

        @pl.when(t < NPAIRS - 1)
        def _():
            pltpu.async_copy(z_hbm.at[sidx_a], rows_a, sem_a)

        pltpu.sync_copy(rows_b, acc.at[lidx_b.at[0]], add=True)
        return 0

    lax.fori_loop(0, NPAIRS, pair, 0)
    plsc.subcore_barrier()

    def bounce(off, nrows):
        pltpu.sync_copy(acc.at[pl.ds(off, nrows)], rows_a.at[pl.ds(0, nrows)])
        pltpu.sync_copy(rows_a.at[pl.ds(0, nrows)],
                        out_hbm.at[pl.ds(base + off, nrows)])

    def writeout(off0, nrows):
        for k in range(nrows // CH):
            bounce(off0 + k * CH, CH)
        if nrows % CH:
            bounce(off0 + (nrows // CH) * CH, nrows % CH)

    @pl.when(sid < NS - 1)
    def _():
        writeout(sid * SLAB, SLAB)

    @pl.when(sid == NS - 1)
    def _():
        writeout((NS - 1) * SLAB, TAIL)


_prop_call = functools.partial(
    pl.kernel,
    out_type=jax.ShapeDtypeStruct((N, D), jnp.float32),
    mesh=_mesh,
    compiler_params=_sc_params,
    scratch_types=[
        pltpu.VMEM((CH,), jnp.int32),
        pltpu.VMEM((CH,), jnp.int32),
        pltpu.VMEM((CH,), jnp.int32),
        pltpu.VMEM((CH,), jnp.int32),
        pltpu.VMEM((K, 128), jnp.int32),
        pltpu.VMEM((K, 128), jnp.int32),
        pltpu.VMEM((CH, D), jnp.float32),
        pltpu.VMEM((CH, D), jnp.float32),
        pltpu.VMEM_SHARED((ACC_ROWS, D), jnp.float32),
        pltpu.SemaphoreType.DMA,
        pltpu.SemaphoreType.DMA,
    ],
)(_prop_body)


BR = 5000


def _scale_body(deg_ref, x_ref, dis_ref, z_ref):
    deg = deg_ref[...]
    dis = jnp.where(deg > 0.0, lax.rsqrt(jnp.maximum(deg, 1e-12)), 0.0)
    dis_ref[...] = dis
    z_ref[...] = dis * x_ref[...]


def _scale_call(deg2, x):
    return pl.pallas_call(
        _scale_body,
        grid=(N // BR,),
        in_specs=[
            pl.BlockSpec((BR, 1), lambda i: (i, 0)),
            pl.BlockSpec((BR, D), lambda i: (i, 0)),
        ],
        out_specs=[
            pl.BlockSpec((BR, 1), lambda i: (i, 0)),
            pl.BlockSpec((BR, D), lambda i: (i, 0)),
        ],
        out_shape=[
            jax.ShapeDtypeStruct((N, 1), jnp.float32),
            jax.ShapeDtypeStruct((N, D), jnp.float32),
        ],
    )(deg2, x)


def _layer_body(s_ref, dis_ref, acc_ref, accout_ref, z_ref):
    dis = dis_ref[...]
    xp = dis * s_ref[...]
    accout_ref[...] = acc_ref[...] + xp
    z_ref[...] = dis * xp


def _layer_call(s, dis, acc):
    return pl.pallas_call(
        _layer_body,
        grid=(N // BR,),
        in_specs=[
            pl.BlockSpec((BR, D), lambda i: (i, 0)),
            pl.BlockSpec((BR, 1), lambda i: (i, 0)),
            pl.BlockSpec((BR, D), lambda i: (i, 0)),
        ],
        out_specs=[
            pl.BlockSpec((BR, D), lambda i: (i, 0)),
            pl.BlockSpec((BR, D), lambda i: (i, 0)),
        ],
        out_shape=[
            jax.ShapeDtypeStruct((N, D), jnp.float32),
            jax.ShapeDtypeStruct((N, D), jnp.float32),
        ],
    )(s, dis, acc)


def _last_body(s_ref, dis_ref, acc_ref, out_ref):
    xp = dis_ref[...] * s_ref[...]
    out_ref[...] = (acc_ref[...] + xp) * (1.0 / (NUM_LAYERS + 1))


def _last_call(s, dis, acc):
    return pl.pallas_call(
        _last_body,
        grid=(N // BR,),
        in_specs=[
            pl.BlockSpec((BR, D), lambda i: (i, 0)),
            pl.BlockSpec((BR, 1), lambda i: (i, 0)),
            pl.BlockSpec((BR, D), lambda i: (i, 0)),
        ],
        out_specs=pl.BlockSpec((BR, D), lambda i: (i, 0)),
        out_shape=jax.ShapeDtypeStruct((N, D), jnp.float32),
    )(s, dis, acc)


def kernel(edge_index, embedding_weight):
    src = edge_index[0]
    dst = edge_index[1]
    pad = E_PAD - E
    src_p = jnp.concatenate([src, jnp.zeros((pad,), jnp.int32)])
    dst_p = jnp.concatenate([dst, jnp.full((pad,), -1, jnp.int32)])

    deg = _deg_call(dst_p)
    dis, z = _scale_call(deg.reshape(N, 1), embedding_weight)

    acc = embedding_weight
    out = None
    for l in range(NUM_LAYERS):
        s = _prop_call(z, src_p, dst_p)
        if l < NUM_LAYERS - 1:
            acc, z = _layer_call(s, dis, acc)
        else:
            out = _last_call(s, dis, acc)
    return out


# speedup vs baseline: 1.6884x; 1.6884x over previous
import functools

import jax
import jax.numpy as jnp
from jax import lax
from jax.experimental import pallas as pl
from jax.experimental.pallas import tpu as pltpu
from jax.experimental.pallas import tpu_sc as plsc

N = 50000
D = 64
NUM_LAYERS = 3
E = 800000

NC = 2
NS = 16
LANES = 16

HALF = N // NC
ACC_ROWS = 25088
DUMP = HALF
SLAB = ACC_ROWS // NS
TAIL = HALF - (NS - 1) * SLAB

CH = 128
CHP = 512
EPT = 50688
NCHUNK_P = EPT // CHP
NCHUNK = EPT // CH
E_PAD = EPT * NS

_mesh = plsc.VectorSubcoreMesh(
    core_axis_name="c", subcore_axis_name="s", num_cores=NC, num_subcores=NS)
_sc_params = pltpu.CompilerParams(use_tc_tiling_on_sc=False)
_sc_params_nl = pltpu.CompilerParams(
    use_tc_tiling_on_sc=False, needs_layout_passes=False)


def _part_body(src_hbm, dst_hbm, esrc_hbm, edst_hbm, cnt_hbm,
               sidx, didx, osrc, odst, cbuf, sem):
    cid = lax.axis_index("c")
    sid = lax.axis_index("s")
    base = cid * HALF
    ebase = sid * EPT
    lane = lax.iota(jnp.int32, LANES)

    def chunk(j, cnt):
        off = ebase + j * CHP
        pltpu.sync_copy(src_hbm.at[pl.ds(off, CHP)], sidx)
        pltpu.sync_copy(dst_hbm.at[pl.ds(off, CHP)], didx)

        def group(g, cnt):
            s = sidx[pl.ds(g * LANES, LANES)]
            d = didx[pl.ds(g * LANES, LANES)]
            loc = d - base
            ok = (loc >= 0) & (loc < HALF)
            pos = plsc.cumsum(ok.astype(jnp.int32))
            idx = cnt + pos - 1
            plsc.store_scatter(osrc, [idx], s, mask=ok)
            plsc.store_scatter(odst, [idx], loc, mask=ok)
            return cnt + jnp.max(pos)

        return lax.fori_loop(0, CHP // LANES, group, cnt)

    cnt = lax.fori_loop(0, NCHUNK_P, chunk, jnp.int32(0))

    target = ((cnt + 2 * CH - 1) // (2 * CH)) * (2 * CH)
    npad = (target - cnt + LANES - 1) // LANES
    dsrc = jnp.zeros((LANES,), jnp.int32)
    ddst = DUMP + lane

    def padg(i, _):
        osrc[pl.ds(cnt + i * LANES, LANES)] = dsrc
        odst[pl.ds(cnt + i * LANES, LANES)] = ddst
        return 0

    lax.fori_loop(0, npad, padg, 0)

    cbuf[pl.ds(0, LANES)] = jnp.full((LANES,), target // CH, jnp.int32)
    pltpu.sync_copy(osrc.at[pl.ds(0, EPT)], esrc_hbm.at[cid, sid])
    pltpu.sync_copy(odst.at[pl.ds(0, EPT)], edst_hbm.at[cid, sid])
    pltpu.sync_copy(cbuf, cnt_hbm.at[cid, sid])


_part_call = functools.partial(
    pl.kernel,
    out_type=(
        jax.ShapeDtypeStruct((NC, NS, EPT), jnp.int32),
        jax.ShapeDtypeStruct((NC, NS, EPT), jnp.int32),
        jax.ShapeDtypeStruct((NC, NS, LANES), jnp.int32),
    ),
    mesh=_mesh,
    compiler_params=_sc_params_nl,
    scratch_types=[
        pltpu.VMEM((CHP,), jnp.int32),
        pltpu.VMEM((CHP,), jnp.int32),
        pltpu.VMEM((EPT + LANES,), jnp.int32),
        pltpu.VMEM((EPT + LANES,), jnp.int32),
        pltpu.VMEM((LANES,), jnp.int32),
        pltpu.SemaphoreType.DMA,
    ],
)(_part_body)


def _deg_body(edst_hbm, cnt_hbm, deg_hbm, lidx, ones_v, zrow, cbuf, dacc, sem):
    cid = lax.axis_index("c")
    sid = lax.axis_index("s")
    base = cid * HALF

    ov = jnp.ones((LANES,), jnp.float32)
    for v in range(CH // LANES):
        ones_v[pl.ds(v * LANES, LANES)] = ov
    zv = jnp.zeros((LANES,), jnp.float32)

    def zbody(i, _):
        zrow[pl.ds(i * LANES, LANES)] = zv
        return 0

    lax.fori_loop(0, SLAB // LANES + 1, zbody, 0, unroll=4)
    pltpu.sync_copy(zrow.at[pl.ds(0, SLAB)], dacc.at[pl.ds(sid * SLAB, SLAB)])
    plsc.subcore_barrier()

    pltpu.sync_copy(cnt_hbm.at[cid, sid], cbuf)
    ncnk = cbuf[...][0]

    def chunk(j, _):
        pltpu.sync_copy(edst_hbm.at[cid, sid, j], lidx.at[0])
        pltpu.sync_copy(ones_v, dacc.at[lidx.at[0]], add=True)
        return 0

    lax.fori_loop(0, ncnk, chunk, 0)
    plsc.subcore_barrier()

    @pl.when(sid < NS - 1)
    def _():
        pltpu.sync_copy(dacc.at[pl.ds(sid * SLAB, SLAB)],
                        zrow.at[pl.ds(0, SLAB)])
        pltpu.sync_copy(zrow.at[pl.ds(0, SLAB)],
                        deg_hbm.at[pl.ds(base + sid * SLAB, SLAB)])

    @pl.when(sid == NS - 1)
    def _():
        pltpu.sync_copy(dacc.at[pl.ds((NS - 1) * SLAB, TAIL)],
                        zrow.at[pl.ds(0, TAIL)])
        pltpu.sync_copy(zrow.at[pl.ds(0, TAIL)],
                        deg_hbm.at[pl.ds(base + (NS - 1) * SLAB, TAIL)])


_deg_call = functools.partial(
    pl.kernel,
    out_type=jax.ShapeDtypeStruct((N,), jnp.float32),
    mesh=_mesh,
    compiler_params=_sc_params,
    scratch_types=[
        pltpu.VMEM((1, CH), jnp.int32),
        pltpu.VMEM((CH,), jnp.float32),
        pltpu.VMEM((SLAB + LANES,), jnp.float32),
        pltpu.VMEM((LANES,), jnp.int32),
        pltpu.VMEM_SHARED((ACC_ROWS,), jnp.float32),
        pltpu.SemaphoreType.DMA,
    ],
)(_deg_body)


def _prop_body(z_hbm, esrc_hbm, edst_hbm, cnt_hbm, out_hbm,
               sidx_a, sidx_b, lidx_a, lidx_b, rows_a, rows_b, cbuf, acc,
               sem_a, sem_b):
    cid = lax.axis_index("c")
    sid = lax.axis_index("s")
    base = cid * HALF

    zv = jnp.zeros((LANES,), jnp.float32)

    def zbody(i, _):
        for c in range(D // LANES):
            rows_a[i, pl.ds(c * LANES, LANES)] = zv
        return 0

    lax.fori_loop(0, CH, zbody, 0, unroll=4)
    for k in range(SLAB // CH):
        pltpu.sync_copy(rows_a, acc.at[pl.ds(sid * SLAB + k * CH, CH)])
    rem = SLAB - (SLAB // CH) * CH
    if rem:
        pltpu.sync_copy(rows_a.at[pl.ds(0, rem)],
                        acc.at[pl.ds(sid * SLAB + (SLAB // CH) * CH, rem)])
    plsc.subcore_barrier()

    pltpu.sync_copy(cnt_hbm.at[cid, sid], cbuf)
    npairs = cbuf[...][0] // 2

    def load_idx(j, sidx, lidx):
        pltpu.sync_copy(esrc_hbm.at[cid, sid, j], sidx)
        pltpu.sync_copy(edst_hbm.at[cid, sid, j], lidx.at[0])

    @pl.when(npairs > 0)
    def _():
        load_idx(0, sidx_a, lidx_a)
        pltpu.async_copy(z_hbm.at[sidx_a], rows_a, sem_a)

    def pair(t, _):
        j0 = 2 * t
        load_idx(j0 + 1, sidx_b, lidx_b)
        pltpu.make_async_copy(z_hbm.at[sidx_a], rows_a, sem_a).wait()
        pltpu.async_copy(z_hbm.at[sidx_b], rows_b, sem_b)
        pltpu.sync_copy(rows_a, acc.at[lidx_a.at[0]], add=True)

        @pl.when(t < npairs - 1)
        def _():
            load_idx(j0 + 2, sidx_a, lidx_a)

        pltpu.make_async_copy(z_hbm.at[sidx_b], rows_b, sem_b).wait()

        @pl.when(t < npairs - 1)
        def _():
            pltpu.async_copy(z_hbm.at[sidx_a], rows_a, sem_a)

        pltpu.sync_copy(rows_b, acc.at[lidx_b.at[0]], add=True)
        return 0

    lax.fori_loop(0, npairs, pair, 0)
    plsc.subcore_barrier()

    def bounce(off, nrows):
        pltpu.sync_copy(acc.at[pl.ds(off, nrows)], rows_a.at[pl.ds(0, nrows)])
        pltpu.sync_copy(rows_a.at[pl.ds(0, nrows)],
                        out_hbm.at[pl.ds(base + off, nrows)])

    def writeout(off0, nrows):
        for k in range(nrows // CH):
            bounce(off0 + k * CH, CH)
        if nrows % CH:
            bounce(off0 + (nrows // CH) * CH, nrows % CH)

    @pl.when(sid < NS - 1)
    def _():
        writeout(sid * SLAB, SLAB)

    @pl.when(sid == NS - 1)
    def _():
        writeout((NS - 1) * SLAB, TAIL)


_prop_call = functools.partial(
    pl.kernel,
    out_type=jax.ShapeDtypeStruct((N, D), jnp.float32),
    mesh=_mesh,
    compiler_params=_sc_params,
    scratch_types=[
        pltpu.VMEM((CH,), jnp.int32),
        pltpu.VMEM((CH,), jnp.int32),
        pltpu.VMEM((1, CH), jnp.int32),
        pltpu.VMEM((1, CH), jnp.int32),
        pltpu.VMEM((CH, D), jnp.float32),
        pltpu.VMEM((CH, D), jnp.float32),
        pltpu.VMEM((LANES,), jnp.int32),
        pltpu.VMEM_SHARED((ACC_ROWS, D), jnp.float32),
        pltpu.SemaphoreType.DMA,
        pltpu.SemaphoreType.DMA,
    ],
)(_prop_body)


BR = 5000


def _scale_body(deg_ref, x_ref, dis_ref, z_ref):
    deg = deg_ref[...]
    dis = jnp.where(deg > 0.0, lax.rsqrt(jnp.maximum(deg, 1e-12)), 0.0)
    dis_ref[...] = dis
    z_ref[...] = dis * x_ref[...]


def _scale_call(deg2, x):
    return pl.pallas_call(
        _scale_body,
        grid=(N // BR,),
        in_specs=[
            pl.BlockSpec((BR, 1), lambda i: (i, 0)),
            pl.BlockSpec((BR, D), lambda i: (i, 0)),
        ],
        out_specs=[
            pl.BlockSpec((BR, 1), lambda i: (i, 0)),
            pl.BlockSpec((BR, D), lambda i: (i, 0)),
        ],
        out_shape=[
            jax.ShapeDtypeStruct((N, 1), jnp.float32),
            jax.ShapeDtypeStruct((N, D), jnp.float32),
        ],
    )(deg2, x)


def _layer_body(s_ref, dis_ref, acc_ref, accout_ref, z_ref):
    dis = dis_ref[...]
    xp = dis * s_ref[...]
    accout_ref[...] = acc_ref[...] + xp
    z_ref[...] = dis * xp


def _layer_call(s, dis, acc):
    return pl.pallas_call(
        _layer_body,
        grid=(N // BR,),
        in_specs=[
            pl.BlockSpec((BR, D), lambda i: (i, 0)),
            pl.BlockSpec((BR, 1), lambda i: (i, 0)),
            pl.BlockSpec((BR, D), lambda i: (i, 0)),
        ],
        out_specs=[
            pl.BlockSpec((BR, D), lambda i: (i, 0)),
            pl.BlockSpec((BR, D), lambda i: (i, 0)),
        ],
        out_shape=[
            jax.ShapeDtypeStruct((N, D), jnp.float32),
            jax.ShapeDtypeStruct((N, D), jnp.float32),
        ],
    )(s, dis, acc)


def _last_body(s_ref, dis_ref, acc_ref, out_ref):
    xp = dis_ref[...] * s_ref[...]
    out_ref[...] = (acc_ref[...] + xp) * (1.0 / (NUM_LAYERS + 1))


def _last_call(s, dis, acc):
    return pl.pallas_call(
        _last_body,
        grid=(N // BR,),
        in_specs=[
            pl.BlockSpec((BR, D), lambda i: (i, 0)),
            pl.BlockSpec((BR, 1), lambda i: (i, 0)),
            pl.BlockSpec((BR, D), lambda i: (i, 0)),
        ],
        out_specs=pl.BlockSpec((BR, D), lambda i: (i, 0)),
        out_shape=jax.ShapeDtypeStruct((N, D), jnp.float32),
    )(s, dis, acc)


def kernel(edge_index, embedding_weight):
    src = edge_index[0]
    dst = edge_index[1]
    pad = E_PAD - E
    src_p = jnp.concatenate([src, jnp.zeros((pad,), jnp.int32)])
    dst_p = jnp.concatenate([dst, jnp.full((pad,), -1, jnp.int32)])

    esrc, edst, cnts = _part_call(src_p, dst_p)
    esrc4 = esrc.reshape(NC, NS, NCHUNK, CH)
    edst4 = edst.reshape(NC, NS, NCHUNK, CH)

    deg = _deg_call(edst4, cnts)
    dis, z = _scale_call(deg.reshape(N, 1), embedding_weight)

    acc = embedding_weight
    out = None
    for l in range(NUM_LAYERS):
        s = _prop_call(z, esrc4, edst4, cnts)
        if l < NUM_LAYERS - 1:
            acc, z = _layer_call(s, dis, acc)
        else:
            out = _last_call(s, dis, acc)
    return out
